# Initial kernel scaffold; baseline (speedup 1.0000x reference)
import jax
import jax.numpy as jnp
from jax.experimental import pallas as pl


def kernel(input_ids, table, gamma, beta):
    raise NotImplementedError("write your pallas kernel here")


# speedup vs baseline: 3.5468x; 3.5468x over previous
import functools

import jax
import jax.numpy as jnp
from jax import lax
from jax.experimental import pallas as pl
from jax.experimental.pallas import tpu as pltpu
from jax.experimental.pallas import tpu_sc as plsc

_B, _L = 16384, 200
_V, _D = 100, 10
_R = _B - 2
_EPS = 1e-5

_NC, _NS = 2, 16
_NW = _NC * _NS

_ROWS_PER_CHUNK = 8
_IDXC = _ROWS_PER_CHUNK * _L
_OUTC = _IDXC * _D
_FULL_CHUNKS = _R // _ROWS_PER_CHUNK
_TAIL_ROWS = _R - _FULL_CHUNKS * _ROWS_PER_CHUNK
_TAIL_IDX = _TAIL_ROWS * _L
_TAIL_OUT = _TAIL_IDX * _D
_ITERS = -(-_FULL_CHUNKS // _NW)

_TABLE_PAD = 1024


def _ln_table_body(table_ref, gamma_ref, beta_ref, out_ref):
    t = table_ref[...]
    mean = jnp.mean(t, axis=1, keepdims=True)
    var = jnp.mean(jnp.square(t - mean), axis=1, keepdims=True)
    normed = (t - mean) * lax.rsqrt(var + _EPS)
    out_ref[...] = normed * gamma_ref[...] + beta_ref[...]


def _ln_table(table, gamma, beta):
    return pl.pallas_call(
        _ln_table_body,
        out_shape=jax.ShapeDtypeStruct((_V, _D), jnp.float32),
    )(table, gamma.reshape(1, _D), beta.reshape(1, _D))


def _expand_groups(idx_v, table_v, out_v, n_groups):
    lane = lax.broadcasted_iota(jnp.int32, (16,), 0)
    pos0 = lane * _D

    def g_body(g, carry):
        idxv = plsc.load_gather(idx_v, [g * 16 + lane])
        srcb = idxv * _D
        posb = pos0 + g * (16 * _D)
        for j in range(_D):
            vals = plsc.load_gather(table_v, [srcb + j])
            plsc.store_scatter(out_v, [posb + j], vals)
        return carry

    lax.fori_loop(0, n_groups, g_body, 0)


def _sc_gather_kernel(ids_hbm, nt_hbm, out_hbm, table_v, idx_v, out_v):
    w = lax.axis_index("s") * _NC + lax.axis_index("c")
    pltpu.sync_copy(nt_hbm, table_v)

    def body(t, carry):
        c = w + t * _NW

        @pl.when(c < _FULL_CHUNKS)
        def _():
            ids_off = _L + c * _IDXC
            pltpu.sync_copy(ids_hbm.at[pl.ds(ids_off, _IDXC)], idx_v)
            _expand_groups(idx_v, table_v, out_v, _IDXC // 16)
            pltpu.sync_copy(out_v, out_hbm.at[pl.ds(c * _OUTC, _OUTC)])

        return carry

    lax.fori_loop(0, _ITERS, body, 0)

    @pl.when(w == 0)
    def _():
        ids_off = _L + _FULL_CHUNKS * _IDXC
        pltpu.sync_copy(ids_hbm.at[pl.ds(ids_off, _TAIL_IDX)],
                        idx_v.at[pl.ds(0, _TAIL_IDX)])
        _expand_groups(idx_v, table_v, out_v, _TAIL_IDX // 16)
        pltpu.sync_copy(out_v.at[pl.ds(0, _TAIL_OUT)],
                        out_hbm.at[pl.ds(_FULL_CHUNKS * _OUTC, _TAIL_OUT)])


@functools.cache
def _sc_gather():
    return pl.kernel(
        _sc_gather_kernel,
        out_type=jax.ShapeDtypeStruct((_R * _L * _D,), jnp.float32),
        mesh=plsc.VectorSubcoreMesh(core_axis_name="c", subcore_axis_name="s"),
        scratch_types=[
            pltpu.VMEM((_TABLE_PAD,), jnp.float32),
            pltpu.VMEM((_IDXC,), jnp.int32),
            pltpu.VMEM((_OUTC,), jnp.float32),
        ],
        compiler_params=pltpu.CompilerParams(needs_layout_passes=False),
    )


def kernel(input_ids, table, gamma, beta):
    nt = _ln_table(table.astype(jnp.float32), gamma.astype(jnp.float32),
                   beta.astype(jnp.float32))
    nt_flat = jnp.concatenate(
        [nt.reshape(-1), jnp.zeros((_TABLE_PAD - _V * _D,), jnp.float32)])
    ids_flat = input_ids.reshape(-1).astype(jnp.int32)
    out_flat = _sc_gather()(ids_flat, nt_flat)
    return out_flat.reshape(_R, _L, _D)


# speedup vs baseline: 37.8063x; 10.6592x over previous
import functools

import jax
import jax.numpy as jnp
from jax import lax
from jax.experimental import pallas as pl
from jax.experimental.pallas import tpu as pltpu
from jax.experimental.pallas import tpu_sc as plsc

_B, _L = 16384, 200
_V, _D = 100, 10
_R = _B - 2
_EPS = 1e-5

_NC, _NS = 2, 16
_NW = _NC * _NS

_CB = 256
_NCHUNK = _B // _CB
_LH = _L // 8
_BT = _B // 128
_SLAB = 2 * 8 * 128
_DSTRIDE = _LH * _BT * 8 * 128
_LHSTRIDE = _BT * 8 * 128
_PHYS = _D * _DSTRIDE
_TABLE_PAD = 1024


def _ln_table_body(table_ref, gamma_ref, beta_ref, out_ref):
    t = table_ref[...]
    mean = jnp.mean(t, axis=1, keepdims=True)
    var = jnp.mean(jnp.square(t - mean), axis=1, keepdims=True)
    normed = (t - mean) * lax.rsqrt(var + _EPS)
    out_ref[...] = normed * gamma_ref[...] + beta_ref[...]


def _ln_table(table, gamma, beta):
    return pl.pallas_call(
        _ln_table_body,
        out_shape=jax.ShapeDtypeStruct((_V, _D), jnp.float32),
    )(table, gamma.reshape(1, _D), beta.reshape(1, _D))


def _sc_gather_kernel(ids_hbm, nt_hbm, out_hbm, table_v, raw_v, idst_v,
                      slab_v, sem):
    w = lax.axis_index("s") * _NC + lax.axis_index("c")
    pltpu.sync_copy(nt_hbm, table_v)
    lane = lax.broadcasted_iota(jnp.int32, (16,), 0)
    lane_l = lane * _L

    def slab_dma(buf, d, lh, c):
        dst = d * _DSTRIDE + lh * _LHSTRIDE + c * (2 * 1024)
        return pltpu.make_async_copy(
            slab_v.at[pl.ds(buf * (_D * _SLAB) + d * _SLAB, _SLAB)],
            out_hbm.at[pl.ds(dst, _SLAB)], sem)

    def do_chunk(c):
        for h in range(2):
            off = (c * _CB + 1 + h * 128) * _L
            if h == 0:
                pltpu.sync_copy(ids_hbm.at[pl.ds(off, 128 * _L)], raw_v)
            else:
                @pl.when(c < _NCHUNK - 1)
                def _():
                    pltpu.sync_copy(ids_hbm.at[pl.ds(off, 128 * _L)], raw_v)

                @pl.when(c == _NCHUNK - 1)
                def _():
                    pltpu.sync_copy(ids_hbm.at[pl.ds(off, 126 * _L)],
                                    raw_v.at[pl.ds(0, 126 * _L)])

            def tbody(l, carry):
                for bv in range(8):
                    v = plsc.load_gather(raw_v, [lane_l + (bv * 16 * _L + l)])
                    v = jnp.minimum(jnp.maximum(v, 0), _V - 1)
                    idst_v[pl.ds(l * _CB + h * 128 + bv * 16, 16)] = v
                return carry

            lax.fori_loop(0, _L, tbody, 0)

        def mbody(lh, carry):
            buf = lax.rem(lh, 2)

            @pl.when(lh >= 2)
            def _():
                for d in range(_D):
                    slab_dma(buf, d, lh - 2, c).wait()

            def lbody(ll, carry2):
                l = lh * 8 + ll
                for bv in range(16):
                    idxv = idst_v[pl.ds(l * _CB + bv * 16, 16)]
                    pos = (bv // 8) * 1024 + ll * 128 + (bv % 8) * 16
                    for d in range(_D):
                        vals = plsc.load_gather(table_v, [idxv + d * _V])
                        slab_v[pl.ds(buf * (_D * _SLAB) + d * _SLAB + pos,
                                     16)] = vals
                return carry2

            lax.fori_loop(0, 8, lbody, 0)
            for d in range(_D):
                slab_dma(buf, d, lh, c).start()
            return carry

        lax.fori_loop(0, _LH, mbody, 0)
        for lh in (_LH - 2, _LH - 1):
            for d in range(_D):
                slab_dma(lh % 2, d, lh, c).wait()

    for t in range(2):
        do_chunk(w * 2 + t)


@functools.cache
def _sc_gather():
    return pl.kernel(
        _sc_gather_kernel,
        out_type=jax.ShapeDtypeStruct((_PHYS,), jnp.float32),
        mesh=plsc.VectorSubcoreMesh(core_axis_name="c", subcore_axis_name="s"),
        scratch_types=[
            pltpu.VMEM((_TABLE_PAD,), jnp.float32),
            pltpu.VMEM((128 * _L,), jnp.int32),
            pltpu.VMEM((_L * _CB,), jnp.int32),
            pltpu.VMEM((2 * _D * _SLAB,), jnp.float32),
            pltpu.SemaphoreType.DMA,
        ],
        compiler_params=pltpu.CompilerParams(needs_layout_passes=False),
    )


def kernel(input_ids, table, gamma, beta):
    nt = _ln_table(table.astype(jnp.float32), gamma.astype(jnp.float32),
                   beta.astype(jnp.float32))
    ntt_flat = jnp.concatenate(
        [nt.T.reshape(-1), jnp.zeros((_TABLE_PAD - _V * _D,), jnp.float32)])
    ids_flat = input_ids.reshape(-1).astype(jnp.int32)
    out_flat = _sc_gather()(ids_flat, ntt_flat)
    out = (out_flat.reshape(_D, _LH, _BT, 8, 128)
           .transpose(2, 4, 1, 3, 0).reshape(_B, _L, _D))
    return out[:_R]


# speedup vs baseline: 69.2975x; 1.8330x over previous
import functools

import jax
import jax.numpy as jnp
from jax import lax
from jax.experimental import pallas as pl
from jax.experimental.pallas import tpu as pltpu
from jax.experimental.pallas import tpu_sc as plsc

_B, _L = 16384, 200
_V, _D = 100, 10
_R = _B - 2
_EPS = 1e-5

_NC, _NS = 2, 16
_NW = _NC * _NS

_CB = 256
_NCHUNK = _B // _CB
_LH = _L // 8
_BT = _B // 128
_SLAB = 2 * 8 * 128
_DSTRIDE = _LH * _BT * 8 * 128
_LHSTRIDE = _BT * 8 * 128
_PHYS = _D * _DSTRIDE
_TABLE_PAD = 1024


def _ln_table_body(table_ref, gamma_ref, beta_ref, out_ref):
    t = table_ref[...]
    mean = jnp.mean(t, axis=1, keepdims=True)
    var = jnp.mean(jnp.square(t - mean), axis=1, keepdims=True)
    normed = (t - mean) * lax.rsqrt(var + _EPS)
    out_ref[...] = normed * gamma_ref[...] + beta_ref[...]


def _ln_table(table, gamma, beta):
    return pl.pallas_call(
        _ln_table_body,
        out_shape=jax.ShapeDtypeStruct((_V, _D), jnp.float32),
    )(table, gamma.reshape(1, _D), beta.reshape(1, _D))


def _sc_gather_kernel(ids_hbm, nt_hbm, out_hbm, table_v, raw_v, idst_v,
                      slab_v, sem):
    w = lax.axis_index("s") * _NC + lax.axis_index("c")
    pltpu.sync_copy(nt_hbm, table_v)
    lane = lax.broadcasted_iota(jnp.int32, (16,), 0)
    lane_l = lane * _L

    def slab_dma(buf, d, lh, c):
        dst = d * _DSTRIDE + lh * _LHSTRIDE + c * (2 * 1024)
        return pltpu.make_async_copy(
            slab_v.at[pl.ds(buf * (_D * _SLAB) + d * _SLAB, _SLAB)],
            out_hbm.at[pl.ds(dst, _SLAB)], sem)

    def do_chunk(c):
        for h in range(2):
            off = (c * _CB + 1 + h * 128) * _L
            if h == 0:
                pltpu.sync_copy(ids_hbm.at[pl.ds(off, 128 * _L)], raw_v)
            else:
                @pl.when(c < _NCHUNK - 1)
                def _():
                    pltpu.sync_copy(ids_hbm.at[pl.ds(off, 128 * _L)], raw_v)

                @pl.when(c == _NCHUNK - 1)
                def _():
                    pltpu.sync_copy(ids_hbm.at[pl.ds(off, 126 * _L)],
                                    raw_v.at[pl.ds(0, 126 * _L)])

            @plsc.parallel_loop(0, _L)
            def tbody(l):
                for bv in range(8):
                    v = plsc.load_gather(raw_v, [lane_l + (bv * 16 * _L + l)])
                    v = plsc.bitcast(
                        jnp.minimum(plsc.bitcast(v, jnp.uint32),
                                    jnp.uint32(_V - 1)), jnp.int32)
                    idst_v[pl.ds(l * _CB + h * 128 + bv * 16, 16)] = v

        def mbody(lh, carry):
            buf = lax.rem(lh, 2)

            @pl.when(lh >= 2)
            def _():
                for d in range(_D):
                    slab_dma(buf, d, lh - 2, c).wait()

            @plsc.parallel_loop(0, 8)
            def lbody(ll):
                l = lh * 8 + ll
                for bv in range(16):
                    idxv = idst_v[pl.ds(l * _CB + bv * 16, 16)]
                    pos = (bv // 8) * 1024 + ll * 128 + (bv % 8) * 16
                    for d in range(_D):
                        vals = plsc.load_gather(table_v, [idxv + d * _V])
                        slab_v[pl.ds(buf * (_D * _SLAB) + d * _SLAB + pos,
                                     16)] = vals
            for d in range(_D):
                slab_dma(buf, d, lh, c).start()
            return carry

        lax.fori_loop(0, _LH, mbody, 0)
        for lh in (_LH - 2, _LH - 1):
            for d in range(_D):
                slab_dma(lh % 2, d, lh, c).wait()

    for t in range(2):
        do_chunk(w * 2 + t)


@functools.cache
def _sc_gather():
    return pl.kernel(
        _sc_gather_kernel,
        out_type=jax.ShapeDtypeStruct((_PHYS,), jnp.float32),
        mesh=plsc.VectorSubcoreMesh(core_axis_name="c", subcore_axis_name="s"),
        scratch_types=[
            pltpu.VMEM((_TABLE_PAD,), jnp.float32),
            pltpu.VMEM((128 * _L,), jnp.int32),
            pltpu.VMEM((_L * _CB,), jnp.int32),
            pltpu.VMEM((2 * _D * _SLAB,), jnp.float32),
            pltpu.SemaphoreType.DMA,
        ],
        compiler_params=pltpu.CompilerParams(needs_layout_passes=False),
    )


def kernel(input_ids, table, gamma, beta):
    nt = _ln_table(table.astype(jnp.float32), gamma.astype(jnp.float32),
                   beta.astype(jnp.float32))
    ntt_flat = jnp.concatenate(
        [nt.T.reshape(-1), jnp.zeros((_TABLE_PAD - _V * _D,), jnp.float32)])
    ids_flat = input_ids.reshape(-1).astype(jnp.int32)
    out_flat = _sc_gather()(ids_flat, ntt_flat)
    out = (out_flat.reshape(_D, _LH, _BT, 8, 128)
           .transpose(2, 4, 1, 3, 0).reshape(_B, _L, _D))
    return out[:_R]
